# baseline (speedup 1.0000x reference)
import functools

import jax
import jax.numpy as jnp
from jax import lax
from jax.experimental import pallas as pl
from jax.experimental.pallas import tpu as pltpu
from jax.experimental.pallas import tpu_sc as plsc

LANES = 16
CHUNK = 8
SC_ROWS = 7168
TC_BR = 256


@functools.lru_cache(maxsize=None)
def _build_sc_kernel(B, D, n_workers):
    rows_per_w = B // n_workers
    n_chunks = rows_per_w // CHUNK
    n_outer = n_chunks // 2

    mesh = plsc.VectorSubcoreMesh(core_axis_name="c", subcore_axis_name="s")

    @functools.partial(
        pl.kernel,
        mesh=mesh,
        out_type=jax.ShapeDtypeStruct((n_workers, LANES), jnp.float32),
        scratch_types=[
            pltpu.VMEM((2, CHUNK, D), jnp.float32),
            pltpu.VMEM((2, CHUNK, D), jnp.float32),
            pltpu.VMEM((n_chunks, CHUNK), jnp.int32),
            pltpu.VMEM((LANES,), jnp.float32),
            pltpu.SemaphoreType.DMA,
            pltpu.SemaphoreType.DMA,
            pltpu.SemaphoreType.DMA,
            pltpu.SemaphoreType.DMA,
        ],
    )
    def sc_fn(x_hbm, t_hbm, cent_hbm, out_hbm,
              x_bufs, c_bufs, idx_all, acc_v, sx0, sx1, sc0, sc1):
        nc = 2
        wid = lax.axis_index("s") * nc + lax.axis_index("c")
        row0 = wid * rows_per_w

        pltpu.sync_copy(t_hbm.at[wid], idx_all)

        sx = (sx0, sx1)
        sc = (sc0, sc1)

        def start(chunk, buf):
            pltpu.async_copy(
                x_hbm.at[pl.ds(row0 + chunk * CHUNK, CHUNK)],
                x_bufs.at[buf], sx[buf])
            pltpu.async_copy(
                cent_hbm.at[idx_all.at[chunk]],
                c_bufs.at[buf], sc[buf])

        def wait(chunk, buf):
            pltpu.make_async_copy(
                x_hbm.at[pl.ds(row0, CHUNK)], x_bufs.at[buf], sx[buf]).wait()
            pltpu.make_async_copy(
                cent_hbm.at[idx_all.at[chunk]], c_bufs.at[buf], sc[buf]).wait()

        def accumulate(buf, accs):
            def body(j, accs):
                o = j * LANES
                new = []
                for r in range(CHUNK):
                    d = (x_bufs[buf, r, pl.ds(o, LANES)]
                         - c_bufs[buf, r, pl.ds(o, LANES)])
                    new.append(accs[r] + d * d)
                return tuple(new)
            return lax.fori_loop(0, D // LANES, body, accs)

        zero = jnp.zeros((LANES,), jnp.float32)
        accs0 = (zero,) * CHUNK

        start(0, 0)

        def outer(g, accs):
            ca = 2 * g
            cb = ca + 1
            start(cb, 1)
            wait(ca, 0)
            accs = accumulate(0, accs)

            @pl.when(g < n_outer - 1)
            def _():
                start(ca + 2, 0)

            wait(cb, 1)
            accs = accumulate(1, accs)
            return accs

        accs = lax.fori_loop(0, n_outer, outer, accs0)

        total = accs[0]
        for r in range(1, CHUNK):
            total = total + accs[r]
        acc_v[...] = total
        pltpu.sync_copy(acc_v, out_hbm.at[wid])

    return sc_fn


@functools.lru_cache(maxsize=None)
def _build_tc_kernel(B, D, C, block_off, n_blocks):
    UNROLL = 16

    def body(t_ref, x_ref, c_hbm, out_ref, c_vmem, sem):
        i = pl.program_id(0)

        @pl.when(i == 0)
        def _():
            pltpu.make_async_copy(c_hbm, c_vmem, sem).start()
            pltpu.make_async_copy(c_hbm, c_vmem, sem).wait()
            out_ref[...] = jnp.zeros_like(out_ref)

        def rows(j, accs):
            accs = list(accs)
            ts = [t_ref[0, 0, j * UNROLL + u] for u in range(UNROLL)]
            for u in range(UNROLL):
                r = j * UNROLL + u
                x = x_ref[pl.ds(r, 1), :]
                c = c_vmem[pl.ds(ts[u], 1), :]
                d = x - c
                accs[u % 4] = accs[u % 4] + d * d
            return tuple(accs)

        zero = jnp.zeros((1, D), jnp.float32)
        a0, a1, a2, a3 = lax.fori_loop(
            0, TC_BR // UNROLL, rows, (zero, zero, zero, zero))
        out_ref[...] += (a0 + a1) + (a2 + a3)

    return pl.pallas_call(
        body,
        grid=(n_blocks,),
        in_specs=[
            pl.BlockSpec((1, 1, TC_BR), lambda i: (i + block_off, 0, 0),
                         memory_space=pltpu.SMEM),
            pl.BlockSpec((TC_BR, D), lambda i: (i + block_off, 0)),
            pl.BlockSpec(memory_space=pl.ANY),
        ],
        out_specs=pl.BlockSpec((1, D), lambda i: (0, 0)),
        out_shape=jax.ShapeDtypeStruct((1, D), jnp.float32),
        scratch_shapes=[
            pltpu.VMEM((C, D), jnp.float32),
            pltpu.SemaphoreType.DMA,
        ],
    )


def kernel(inputs, targets, centers):
    B, D = inputs.shape
    C = centers.shape[0]
    t32 = targets.astype(jnp.int32)
    info = plsc.get_sparse_core_info()
    n_workers = info.num_cores * info.num_subcores

    b_sc = SC_ROWS
    b_tc = B - b_sc

    t_sc = t32[:b_sc].reshape(n_workers, b_sc // n_workers // CHUNK, CHUNK)
    sc_part = _build_sc_kernel(b_sc, D, n_workers)(inputs, t_sc, centers)

    t_tc = t32.reshape(B // TC_BR, 1, TC_BR)
    tc_part = _build_tc_kernel(B, D, C, b_sc // TC_BR, b_tc // TC_BR)(
        t_tc, inputs, centers)

    return (jnp.sum(sc_part) + jnp.sum(tc_part)) / B


# speedup vs baseline: 1.0847x; 1.0847x over previous
import functools

import jax
import jax.numpy as jnp
from jax import lax
from jax.experimental import pallas as pl
from jax.experimental.pallas import tpu as pltpu
from jax.experimental.pallas import tpu_sc as plsc

LANES = 16
CHUNK = 8
SC_ROWS = 8192
TC_BR = 256


@functools.lru_cache(maxsize=None)
def _build_sc_kernel(B, D, n_workers):
    rows_per_w = B // n_workers
    n_chunks = rows_per_w // CHUNK
    n_outer = n_chunks // 2

    mesh = plsc.VectorSubcoreMesh(core_axis_name="c", subcore_axis_name="s")

    @functools.partial(
        pl.kernel,
        mesh=mesh,
        out_type=jax.ShapeDtypeStruct((n_workers, LANES), jnp.float32),
        scratch_types=[
            pltpu.VMEM((2, CHUNK, D), jnp.float32),
            pltpu.VMEM((2, CHUNK, D), jnp.float32),
            pltpu.VMEM((rows_per_w,), jnp.int32),
            pltpu.VMEM((LANES,), jnp.float32),
            pltpu.SemaphoreType.DMA,
            pltpu.SemaphoreType.DMA,
            pltpu.SemaphoreType.DMA,
            pltpu.SemaphoreType.DMA,
        ],
    )
    def sc_fn(x_hbm, t_hbm, cent_hbm, out_hbm,
              x_bufs, c_bufs, idx_all, acc_v, sx0, sx1, sc0, sc1):
        nc = 2
        wid = lax.axis_index("s") * nc + lax.axis_index("c")
        row0 = wid * rows_per_w

        pltpu.sync_copy(t_hbm.at[pl.ds(row0, rows_per_w)], idx_all)

        sx = (sx0, sx1)
        sc = (sc0, sc1)

        def start(chunk, buf):
            pltpu.async_copy(
                x_hbm.at[pl.ds(row0 + chunk * CHUNK, CHUNK)],
                x_bufs.at[buf], sx[buf])
            pltpu.async_copy(
                cent_hbm.at[idx_all.at[pl.ds(chunk * CHUNK, CHUNK)]],
                c_bufs.at[buf], sc[buf])

        def wait(chunk, buf):
            pltpu.make_async_copy(
                x_hbm.at[pl.ds(row0, CHUNK)], x_bufs.at[buf], sx[buf]).wait()
            pltpu.make_async_copy(
                cent_hbm.at[idx_all.at[pl.ds(chunk * CHUNK, CHUNK)]],
                c_bufs.at[buf], sc[buf]).wait()

        def accumulate(buf, accs):
            def body(j, accs):
                o = j * LANES
                new = []
                for r in range(CHUNK):
                    d = (x_bufs[buf, r, pl.ds(o, LANES)]
                         - c_bufs[buf, r, pl.ds(o, LANES)])
                    new.append(accs[r] + d * d)
                return tuple(new)
            return lax.fori_loop(0, D // LANES, body, accs)

        zero = jnp.zeros((LANES,), jnp.float32)
        accs0 = (zero,) * CHUNK

        start(0, 0)

        def outer(g, accs):
            ca = 2 * g
            cb = ca + 1
            start(cb, 1)
            wait(ca, 0)
            accs = accumulate(0, accs)

            @pl.when(g < n_outer - 1)
            def _():
                start(ca + 2, 0)

            wait(cb, 1)
            accs = accumulate(1, accs)
            return accs

        accs = lax.fori_loop(0, n_outer, outer, accs0)

        total = accs[0]
        for r in range(1, CHUNK):
            total = total + accs[r]
        acc_v[...] = total
        pltpu.sync_copy(acc_v, out_hbm.at[wid])

    return sc_fn


@functools.lru_cache(maxsize=None)
def _build_tc_kernel(B, D, C, block_off, n_blocks):
    UNROLL = 16

    def body(t_ref, x_ref, c_hbm, out_ref, c_vmem, sem):
        i = pl.program_id(0)

        @pl.when(i == 0)
        def _():
            pltpu.make_async_copy(c_hbm, c_vmem, sem).start()
            pltpu.make_async_copy(c_hbm, c_vmem, sem).wait()
            out_ref[...] = jnp.zeros_like(out_ref)

        def rows(j, accs):
            accs = list(accs)
            ts = [t_ref[j * UNROLL + u] for u in range(UNROLL)]
            for u in range(UNROLL):
                r = j * UNROLL + u
                x = x_ref[pl.ds(r, 1), :]
                c = c_vmem[pl.ds(ts[u], 1), :]
                d = x - c
                accs[u % 4] = accs[u % 4] + d * d
            return tuple(accs)

        zero = jnp.zeros((1, D), jnp.float32)
        a0, a1, a2, a3 = lax.fori_loop(
            0, TC_BR // UNROLL, rows, (zero, zero, zero, zero))
        out_ref[...] += (a0 + a1) + (a2 + a3)

    return pl.pallas_call(
        body,
        grid=(n_blocks,),
        in_specs=[
            pl.BlockSpec((TC_BR,), lambda i: (i + block_off,),
                         memory_space=pltpu.SMEM),
            pl.BlockSpec((TC_BR, D), lambda i: (i + block_off, 0)),
            pl.BlockSpec(memory_space=pl.ANY),
        ],
        out_specs=pl.BlockSpec((1, D), lambda i: (0, 0)),
        out_shape=jax.ShapeDtypeStruct((1, D), jnp.float32),
        scratch_shapes=[
            pltpu.VMEM((C, D), jnp.float32),
            pltpu.SemaphoreType.DMA,
        ],
    )


def kernel(inputs, targets, centers):
    B, D = inputs.shape
    C = centers.shape[0]
    t32 = targets.astype(jnp.int32)
    info = plsc.get_sparse_core_info()
    n_workers = info.num_cores * info.num_subcores

    b_sc = SC_ROWS
    b_tc = B - b_sc

    sc_part = _build_sc_kernel(b_sc, D, n_workers)(inputs, t32, centers)
    tc_part = _build_tc_kernel(B, D, C, b_sc // TC_BR, b_tc // TC_BR)(
        t32, inputs, centers)

    return (jnp.sum(sc_part) + jnp.sum(tc_part)) / B
